# baseline (speedup 1.0000x reference)
import jax
import jax.numpy as jnp
from jax.experimental import pallas as pl

_B = 4
_N = 5000
_NPAD = 5120
_BLKS = _NPAD // 128
_MAXDET = 100
_MINCONF = 0.7
_NMS_T = 0.3


def _nms_kernel(rois_ref, cls_ref, out_ref):
    y1 = rois_ref[:, 0]
    x1 = rois_ref[:, 1]
    y2 = rois_ref[:, 2]
    x2 = rois_ref[:, 3]
    dy = cls_ref[:, 0] * 0.1
    dx = cls_ref[:, 1] * 0.1
    dh = cls_ref[:, 2] * 0.2
    dw = cls_ref[:, 3] * 0.2
    cls_f = cls_ref[:, 4]
    raw_scores = cls_ref[:, 5]

    h = y2 - y1
    w = x2 - x1
    cy = y1 + 0.5 * h + dy * h
    cx = x1 + 0.5 * w + dx * w
    h = h * jnp.exp(dh)
    w = w * jnp.exp(dw)
    ry1 = jnp.clip(cy - 0.5 * h, 0.0, 1.0)
    rx1 = jnp.clip(cx - 0.5 * w, 0.0, 1.0)
    ry2 = jnp.clip((cy - 0.5 * h) + h, 0.0, 1.0)
    rx2 = jnp.clip((cx - 0.5 * w) + w, 0.0, 1.0)

    cls_i = cls_f.astype(jnp.int32)
    keep = (cls_i > 0) & (raw_scores >= _MINCONF)
    scores0 = jnp.where(keep, raw_scores, -1.0)

    off = cls_f * 4.0
    ny1 = ry1 + off
    nx1 = rx1 + off
    ny2 = ry2 + off
    nx2 = rx2 + off
    areas = (ny2 - ny1) * (nx2 - nx1)

    shape = (_B, _BLKS, 128)
    flat_iota = (
        jax.lax.broadcasted_iota(jnp.int32, shape, 1) * 128
        + jax.lax.broadcasted_iota(jnp.int32, shape, 2)
    )
    big = jnp.int32(_NPAD + 1)
    lane2 = jax.lax.broadcasted_iota(jnp.int32, (_B, 128), 1)

    def rmax(x):
        return jnp.max(jnp.max(x, axis=2), axis=1)

    def rmin(x):
        return jnp.min(jnp.min(x, axis=2), axis=1)

    def rsum(x):
        return jnp.sum(jnp.sum(x, axis=2), axis=1)

    def body(i, carry):
        scores, a0, a1, a2, a3, a4, a5 = carry
        best = rmax(scores)
        bestb = best[:, None, None]
        idx = rmin(jnp.where(scores == bestb, flat_iota, big))
        isbest = flat_iota == idx[:, None, None]
        zero = jnp.zeros(shape, jnp.float32)

        def sel(v):
            return rsum(jnp.where(isbest, v, zero))

        sy1 = sel(ry1)
        sx1 = sel(rx1)
        sy2 = sel(ry2)
        sx2 = sel(rx2)
        scls = sel(cls_f)
        soff = scls * 4.0
        by1 = sy1 + soff
        bx1 = sx1 + soff
        by2 = sy2 + soff
        bx2 = sx2 + soff
        barea = (by2 - by1) * (bx2 - bx1)
        valid = best > 0.0

        yy1 = jnp.maximum(by1[:, None, None], ny1)
        xx1 = jnp.maximum(bx1[:, None, None], nx1)
        yy2 = jnp.minimum(by2[:, None, None], ny2)
        xx2 = jnp.minimum(bx2[:, None, None], nx2)
        inter = jnp.maximum(yy2 - yy1, 0.0) * jnp.maximum(xx2 - xx1, 0.0)
        iou = inter / (barea[:, None, None] + areas - inter + 1e-8)
        supp = valid[:, None, None] & ((iou > _NMS_T) | isbest)
        new_scores = jnp.where(supp, -1.0, scores)

        hot = lane2 == i

        def acc(a, v):
            vm = jnp.where(valid, v, 0.0)
            return a + jnp.where(hot, vm[:, None], 0.0)

        return (
            new_scores,
            acc(a0, sy1),
            acc(a1, sx1),
            acc(a2, sy2),
            acc(a3, sx2),
            acc(a4, scls),
            acc(a5, best),
        )

    z = jnp.zeros((_B, 128), jnp.float32)
    res = jax.lax.fori_loop(0, _MAXDET, body, (scores0, z, z, z, z, z, z))
    for k in range(6):
        out_ref[:, k, :] = res[1 + k]


def kernel(rois, classifications):
    rois_t = jnp.transpose(rois, (0, 2, 1))
    cls_t = jnp.transpose(classifications, (0, 2, 1))
    pad = _NPAD - _N
    rois_t = jnp.pad(rois_t, ((0, 0), (0, 0), (0, pad)))
    cls_t = jnp.pad(cls_t, ((0, 0), (0, 0), (0, pad)))
    rois_t = rois_t.reshape(_B, 4, _BLKS, 128)
    cls_t = cls_t.reshape(_B, 6, _BLKS, 128)

    out = pl.pallas_call(
        _nms_kernel,
        out_shape=jax.ShapeDtypeStruct((_B, 6, 128), jnp.float32),
    )(rois_t, cls_t)
    return jnp.transpose(out[:, :, :_MAXDET], (0, 2, 1))
